# baseline (speedup 1.0000x reference)
import functools

import jax
import jax.numpy as jnp
from jax import lax
from jax.experimental import pallas as pl
from jax.experimental.pallas import tpu as pltpu
from jax.experimental.pallas import tpu_sc as plsc

N = 10000
E = 320000
D = 128
H = 64
NCLS = 10

NC = 2
NS = 16
NW = NC * NS

CHUNK = 512
NCH = 20
EPT = NCH * CHUNK
E_PAD = NW * EPT
NACC = 10112
SLICE = NACC // NS

_mesh = plsc.VectorSubcoreMesh(core_axis_name="c", subcore_axis_name="s")
_sc_params = pltpu.CompilerParams(use_tc_tiling_on_sc=False)


DEGW = 16


@functools.partial(
    pl.kernel,
    out_type=jax.ShapeDtypeStruct((NC, NACC, DEGW), jnp.float32),
    mesh=_mesh,
    scratch_types=[
        pltpu.VMEM((NCH, CHUNK), jnp.int32),
        pltpu.VMEM((CHUNK, DEGW), jnp.float32),
        pltpu.VMEM_SHARED((NACC, DEGW), jnp.float32),
        pltpu.SemaphoreType.DMA,
    ],
    compiler_params=_sc_params,
)
def _deg_kernel(dst_hbm, ones_hbm, zeros_hbm, out_hbm, dst_v, ones_v, acc, sem):
    cid = lax.axis_index("c")
    sid = lax.axis_index("s")
    wid = sid * NC + cid
    pltpu.sync_copy(dst_hbm.at[wid], dst_v)
    pltpu.sync_copy(ones_hbm, ones_v)
    pltpu.sync_copy(zeros_hbm.at[pl.ds(sid * SLICE, SLICE)],
                    acc.at[pl.ds(sid * SLICE, SLICE)])
    plsc.subcore_barrier()

    def body(j, carry):
        pltpu.async_copy(ones_v, acc.at[dst_v.at[j]], sem, add=True)
        return carry

    lax.fori_loop(0, NCH, body, 0)

    def drain(j, carry):
        pltpu.make_async_copy(ones_v, acc.at[dst_v.at[j]], sem).wait()
        return carry

    lax.fori_loop(0, NCH, drain, 0)
    plsc.subcore_barrier()
    pltpu.sync_copy(acc.at[pl.ds(sid * SLICE, SLICE)],
                    out_hbm.at[cid, pl.ds(sid * SLICE, SLICE)])


RING = 4
LAG = 2
HH = H // 2


@functools.partial(
    pl.kernel,
    out_type=(jax.ShapeDtypeStruct((NC, NACC, HH), jnp.float32),
              jax.ShapeDtypeStruct((NC, NACC, HH), jnp.float32)),
    mesh=_mesh,
    scratch_types=[
        pltpu.VMEM((NCH, CHUNK), jnp.int32),
        pltpu.VMEM((NCH, CHUNK), jnp.int32),
        pltpu.VMEM((RING, CHUNK, HH), jnp.float32),
        pltpu.VMEM_SHARED((NACC, HH), jnp.float32),
        pltpu.VMEM_SHARED((NACC, HH), jnp.float32),
    ] + [pltpu.SemaphoreType.DMA] * (2 * RING),
    compiler_params=_sc_params,
)
def _agg_kernel(g_lo, g_hi, src_hbm, dst_hbm, zeros_hbm, out_lo, out_hi,
                src_v, dst_v, rows, acc, g_sh, *sems):
    gsem = sems[:RING]
    ssem = sems[RING:]
    cid = lax.axis_index("c")
    sid = lax.axis_index("s")
    wid = sid * NC + cid
    pltpu.sync_copy(src_hbm.at[wid], src_v)
    pltpu.sync_copy(dst_hbm.at[wid], dst_v)

    for g_in, out in ((g_lo, out_lo), (g_hi, out_hi)):
        pltpu.sync_copy(g_in.at[pl.ds(sid * SLICE, SLICE)],
                        g_sh.at[pl.ds(sid * SLICE, SLICE)])
        pltpu.sync_copy(zeros_hbm, acc.at[pl.ds(sid * SLICE, SLICE)])
        plsc.subcore_barrier()

        def g_issue(j, b):
            pltpu.async_copy(g_sh.at[src_v.at[j]], rows.at[b], gsem[b])

        def g_wait(j, b):
            pltpu.make_async_copy(g_sh.at[src_v.at[j]], rows.at[b],
                                  gsem[b]).wait()

        def s_issue(j, b):
            pltpu.async_copy(rows.at[b], acc.at[dst_v.at[j]], ssem[b],
                             add=True)

        def s_wait(j, b):
            pltpu.make_async_copy(rows.at[b], acc.at[dst_v.at[j]],
                                  ssem[b]).wait()

        for b in range(RING):
            g_issue(b, b)
        for j in range(LAG):
            g_wait(j, j)
            s_issue(j, j)

        def body(o, carry):
            for b in range(RING):
                i = o * RING + b
                s_wait(i - RING, b)
                g_issue(i, b)
                bp = (b - LAG) % RING
                g_wait(i - LAG, bp)
                s_issue(i - LAG, bp)
            return carry

        lax.fori_loop(1, NCH // RING, body, 0)

        for j in range(NCH - LAG, NCH):
            b = j % RING
            g_wait(j, b)
            s_issue(j, b)
        for j in range(NCH - RING, NCH):
            s_wait(j, j % RING)

        plsc.subcore_barrier()
        pltpu.sync_copy(acc.at[pl.ds(sid * SLICE, SLICE)],
                        out.at[cid, pl.ds(sid * SLICE, SLICE)])


_EPS = 1e-5


def _tc_norm_mm_body(x_ref, gnw, gnb, gnms, w1_ref, h1_ref):
    x = x_ref[...]
    mean = jnp.mean(x, axis=0, keepdims=True)
    xc = x - mean * gnms[...]
    var = jnp.mean(xc * xc, axis=0, keepdims=True)
    hn = xc * jax.lax.rsqrt(var + _EPS) * gnw[...] + gnb[...]
    h1_ref[...] = jnp.dot(hn, w1_ref[...], preferred_element_type=jnp.float32)


def _tc_scale_body(h1_ref, degp_ref, glo_ref, ghi_ref, dinv_ref):
    deg = degp_ref[0, :N, 0:1] + degp_ref[1, :N, 0:1] + 1.0
    dinv = jax.lax.rsqrt(deg)
    dinv_ref[...] = dinv
    g = h1_ref[...] * dinv
    glo_ref[:N, :] = g[:, :HH]
    glo_ref[N:, :] = jnp.zeros((NACC - N, HH), jnp.float32)
    ghi_ref[:N, :] = g[:, HH:]
    ghi_ref[N:, :] = jnp.zeros((NACC - N, HH), jnp.float32)


def _tc_mid_body(splo_ref, sphi_ref, glo_ref, ghi_ref, dinv_ref, b_ref,
                 gnw, gnb, gnms, w_ref, glo_out, ghi_out):
    s = jnp.concatenate(
        [splo_ref[0, :N, :] + splo_ref[1, :N, :] + glo_ref[:N, :],
         sphi_ref[0, :N, :] + sphi_ref[1, :N, :] + ghi_ref[:N, :]], axis=1)
    dinv = dinv_ref[...]
    h = jnp.maximum(dinv * s + b_ref[...], 0.0)
    mean = jnp.mean(h, axis=0, keepdims=True)
    hc = h - mean * gnms[...]
    var = jnp.mean(hc * hc, axis=0, keepdims=True)
    hn = hc * jax.lax.rsqrt(var + _EPS) * gnw[...] + gnb[...]
    h2 = jnp.dot(hn, w_ref[...], preferred_element_type=jnp.float32)
    g = h2 * dinv
    glo_out[:N, :] = g[:, :HH]
    glo_out[N:, :] = jnp.zeros((NACC - N, HH), jnp.float32)
    ghi_out[:N, :] = g[:, HH:]
    ghi_out[N:, :] = jnp.zeros((NACC - N, HH), jnp.float32)


def _tc_fin_body(splo_ref, sphi_ref, glo_ref, ghi_ref, dinv_ref, b_ref,
                 dw_ref, db_ref, ow_ref, ob_ref, out_ref):
    s = jnp.concatenate(
        [splo_ref[0, :N, :] + splo_ref[1, :N, :] + glo_ref[:N, :],
         sphi_ref[0, :N, :] + sphi_ref[1, :N, :] + ghi_ref[:N, :]], axis=1)
    h = jnp.maximum(dinv_ref[...] * s + b_ref[...], 0.0)
    gm = jnp.mean(h, axis=0, keepdims=True)
    dh = jnp.maximum(
        jnp.dot(gm, dw_ref[...], preferred_element_type=jnp.float32) + db_ref[...],
        0.0)
    logits = jnp.dot(dh, ow_ref[...], preferred_element_type=jnp.float32) + ob_ref[...]
    e = jnp.exp(logits - jnp.max(logits, axis=1, keepdims=True))
    out_ref[...] = e / jnp.sum(e, axis=1, keepdims=True)


_tc_norm_mm = pl.pallas_call(
    _tc_norm_mm_body,
    out_shape=jax.ShapeDtypeStruct((N, H), jnp.float32),
)

_tc_scale = pl.pallas_call(
    _tc_scale_body,
    out_shape=(jax.ShapeDtypeStruct((NACC, HH), jnp.float32),
               jax.ShapeDtypeStruct((NACC, HH), jnp.float32),
               jax.ShapeDtypeStruct((N, 1), jnp.float32)),
)

_tc_mid = pl.pallas_call(
    _tc_mid_body,
    out_shape=(jax.ShapeDtypeStruct((NACC, HH), jnp.float32),
               jax.ShapeDtypeStruct((NACC, HH), jnp.float32)),
)

_tc_fin = pl.pallas_call(
    _tc_fin_body,
    out_shape=jax.ShapeDtypeStruct((1, NCLS), jnp.float32),
)


def _r2(v):
    return v.reshape(1, -1)


def kernel(x, edge_index, gn0_w, gn0_b, gn0_ms, W1, b1, gn1_w, gn1_b, gn1_ms,
           W2, b2, gn2_w, gn2_b, gn2_ms, W3, b3, dense_W, dense_b, out_W, out_b):
    ei = edge_index.astype(jnp.int32)
    pad = E_PAD - E
    src_p = jnp.concatenate([ei[0], jnp.zeros((pad,), jnp.int32)])
    dst_p = jnp.concatenate([ei[1], jnp.full((pad,), N, jnp.int32)])
    src_blk = src_p.reshape(NW, NCH, CHUNK)
    dst_blk = dst_p.reshape(NW, NCH, CHUNK)
    zeros_acc = jnp.zeros((SLICE, HH), jnp.float32)
    ones_deg = jnp.ones((CHUNK, DEGW), jnp.float32)
    zeros_deg = jnp.zeros((NACC, DEGW), jnp.float32)

    degp = _deg_kernel(dst_blk, ones_deg, zeros_deg)

    h1 = _tc_norm_mm(x, _r2(gn0_w), _r2(gn0_b), _r2(gn0_ms), W1)
    g1lo, g1hi, dinv = _tc_scale(h1, degp)
    sp1lo, sp1hi = _agg_kernel(g1lo, g1hi, src_blk, dst_blk, zeros_acc)
    g2lo, g2hi = _tc_mid(sp1lo, sp1hi, g1lo, g1hi, dinv, _r2(b1),
                         _r2(gn1_w), _r2(gn1_b), _r2(gn1_ms), W2)
    sp2lo, sp2hi = _agg_kernel(g2lo, g2hi, src_blk, dst_blk, zeros_acc)
    g3lo, g3hi = _tc_mid(sp2lo, sp2hi, g2lo, g2hi, dinv, _r2(b2),
                         _r2(gn2_w), _r2(gn2_b), _r2(gn2_ms), W3)
    sp3lo, sp3hi = _agg_kernel(g3lo, g3hi, src_blk, dst_blk, zeros_acc)
    return _tc_fin(sp3lo, sp3hi, g3lo, g3hi, dinv, _r2(b3), dense_W,
                   _r2(dense_b), out_W, _r2(out_b))


# speedup vs baseline: 1.1046x; 1.1046x over previous
import functools

import jax
import jax.numpy as jnp
from jax import lax
from jax.experimental import pallas as pl
from jax.experimental.pallas import tpu as pltpu
from jax.experimental.pallas import tpu_sc as plsc

N = 10000
E = 320000
D = 128
H = 64
NCLS = 10

NC = 2
NS = 16
NW = NC * NS

CHUNK = 128
NCH = 80
EPT = NCH * CHUNK
E_PAD = NW * EPT
NACC = 10112
SLICE = NACC // NS

_mesh = plsc.VectorSubcoreMesh(core_axis_name="c", subcore_axis_name="s")
_sc_params = pltpu.CompilerParams(use_tc_tiling_on_sc=False)


DEGW = 16


@functools.partial(
    pl.kernel,
    out_type=jax.ShapeDtypeStruct((NC, NACC, DEGW), jnp.float32),
    mesh=_mesh,
    scratch_types=[
        pltpu.VMEM((NCH, CHUNK), jnp.int32),
        pltpu.VMEM((CHUNK, DEGW), jnp.float32),
        pltpu.VMEM_SHARED((NACC, DEGW), jnp.float32),
        pltpu.SemaphoreType.DMA,
    ],
    compiler_params=_sc_params,
)
def _deg_kernel(dst_hbm, ones_hbm, zeros_hbm, out_hbm, dst_v, ones_v, acc, sem):
    cid = lax.axis_index("c")
    sid = lax.axis_index("s")
    wid = sid * NC + cid
    pltpu.sync_copy(dst_hbm.at[wid], dst_v)
    pltpu.sync_copy(ones_hbm, ones_v)
    pltpu.sync_copy(zeros_hbm.at[pl.ds(sid * SLICE, SLICE)],
                    acc.at[pl.ds(sid * SLICE, SLICE)])
    plsc.subcore_barrier()

    def body(j, carry):
        pltpu.async_copy(ones_v, acc.at[dst_v.at[j]], sem, add=True)
        return carry

    lax.fori_loop(0, NCH, body, 0)

    def drain(j, carry):
        pltpu.make_async_copy(ones_v, acc.at[dst_v.at[j]], sem).wait()
        return carry

    lax.fori_loop(0, NCH, drain, 0)
    plsc.subcore_barrier()
    pltpu.sync_copy(acc.at[pl.ds(sid * SLICE, SLICE)],
                    out_hbm.at[cid, pl.ds(sid * SLICE, SLICE)])


RING = 8
LAG = 4
HH = H // 2


@functools.partial(
    pl.kernel,
    out_type=jax.ShapeDtypeStruct((NC, NACC, H), jnp.float32),
    mesh=_mesh,
    scratch_types=[
        pltpu.VMEM((NCH, CHUNK), jnp.int32),
        pltpu.VMEM((NCH, CHUNK), jnp.int32),
        pltpu.VMEM((RING, CHUNK, HH), jnp.float32),
        pltpu.VMEM_SHARED((NACC, HH), jnp.float32),
        pltpu.VMEM_SHARED((NACC, HH), jnp.float32),
    ] + [pltpu.SemaphoreType.DMA] * (2 * RING),
    compiler_params=_sc_params,
)
def _agg_kernel(g_in, src_hbm, dst_hbm, zeros_hbm, out_hbm,
                src_v, dst_v, rows, acc, g_sh, *sems):
    gsem = sems[:RING]
    ssem = sems[RING:]
    cid = lax.axis_index("c")
    sid = lax.axis_index("s")
    wid = sid * NC + cid
    pltpu.sync_copy(src_hbm.at[wid], src_v)
    pltpu.sync_copy(dst_hbm.at[wid], dst_v)

    for p in range(2):
        pltpu.sync_copy(g_in.at[pl.ds(sid * SLICE, SLICE), pl.ds(p * HH, HH)],
                        g_sh.at[pl.ds(sid * SLICE, SLICE)])
        pltpu.sync_copy(zeros_hbm, acc.at[pl.ds(sid * SLICE, SLICE)])
        plsc.subcore_barrier()

        def g_issue(j, b):
            pltpu.async_copy(g_sh.at[src_v.at[j]], rows.at[b], gsem[b])

        def g_wait(j, b):
            pltpu.make_async_copy(g_sh.at[src_v.at[j]], rows.at[b],
                                  gsem[b]).wait()

        def s_issue(j, b):
            pltpu.async_copy(rows.at[b], acc.at[dst_v.at[j]], ssem[b],
                             add=True)

        def s_wait(j, b):
            pltpu.make_async_copy(rows.at[b], acc.at[dst_v.at[j]],
                                  ssem[b]).wait()

        for b in range(RING):
            g_issue(b, b)
        for j in range(LAG):
            g_wait(j, j)
            s_issue(j, j)

        def body(o, carry):
            for b in range(RING):
                i = o * RING + b
                s_wait(i - RING, b)
                g_issue(i, b)
                bp = (b - LAG) % RING
                g_wait(i - LAG, bp)
                s_issue(i - LAG, bp)
            return carry

        lax.fori_loop(1, NCH // RING, body, 0)

        for j in range(NCH - LAG, NCH):
            b = j % RING
            g_wait(j, b)
            s_issue(j, b)
        for j in range(NCH - RING, NCH):
            s_wait(j, j % RING)

        plsc.subcore_barrier()
        pltpu.sync_copy(acc.at[pl.ds(sid * SLICE, SLICE)],
                        out_hbm.at[cid, pl.ds(sid * SLICE, SLICE),
                                   pl.ds(p * HH, HH)])


_EPS = 1e-5


def _tc_norm_mm_body(x_ref, gnw, gnb, gnms, w1_ref, h1_ref):
    x = x_ref[...]
    mean = jnp.mean(x, axis=0, keepdims=True)
    xc = x - mean * gnms[...]
    var = jnp.mean(xc * xc, axis=0, keepdims=True)
    hn = xc * jax.lax.rsqrt(var + _EPS) * gnw[...] + gnb[...]
    h1_ref[...] = jnp.dot(hn, w1_ref[...], preferred_element_type=jnp.float32)


def _tc_scale_body(h1_ref, degp_ref, g_ref, dinv_ref):
    deg = degp_ref[0, :N, 0:1] + degp_ref[1, :N, 0:1] + 1.0
    dinv = jax.lax.rsqrt(deg)
    dinv_ref[...] = dinv
    g_ref[:N, :] = h1_ref[...] * dinv
    g_ref[N:, :] = jnp.zeros((NACC - N, H), jnp.float32)


def _tc_mid_body(sp_ref, g_ref, dinv_ref, b_ref,
                 gnw, gnb, gnms, w_ref, g_out):
    s = sp_ref[0, :N, :] + sp_ref[1, :N, :] + g_ref[:N, :]
    dinv = dinv_ref[...]
    h = jnp.maximum(dinv * s + b_ref[...], 0.0)
    mean = jnp.mean(h, axis=0, keepdims=True)
    hc = h - mean * gnms[...]
    var = jnp.mean(hc * hc, axis=0, keepdims=True)
    hn = hc * jax.lax.rsqrt(var + _EPS) * gnw[...] + gnb[...]
    h2 = jnp.dot(hn, w_ref[...], preferred_element_type=jnp.float32)
    g_out[:N, :] = h2 * dinv
    g_out[N:, :] = jnp.zeros((NACC - N, H), jnp.float32)


def _tc_fin_body(sp_ref, g_ref, dinv_ref, b_ref,
                 dw_ref, db_ref, ow_ref, ob_ref, out_ref):
    s = sp_ref[0, :N, :] + sp_ref[1, :N, :] + g_ref[:N, :]
    h = jnp.maximum(dinv_ref[...] * s + b_ref[...], 0.0)
    gm = jnp.mean(h, axis=0, keepdims=True)
    dh = jnp.maximum(
        jnp.dot(gm, dw_ref[...], preferred_element_type=jnp.float32) + db_ref[...],
        0.0)
    logits = jnp.dot(dh, ow_ref[...], preferred_element_type=jnp.float32) + ob_ref[...]
    e = jnp.exp(logits - jnp.max(logits, axis=1, keepdims=True))
    out_ref[...] = e / jnp.sum(e, axis=1, keepdims=True)


_tc_norm_mm = pl.pallas_call(
    _tc_norm_mm_body,
    out_shape=jax.ShapeDtypeStruct((N, H), jnp.float32),
)

_tc_scale = pl.pallas_call(
    _tc_scale_body,
    out_shape=(jax.ShapeDtypeStruct((NACC, H), jnp.float32),
               jax.ShapeDtypeStruct((N, 1), jnp.float32)),
)

_tc_mid = pl.pallas_call(
    _tc_mid_body,
    out_shape=jax.ShapeDtypeStruct((NACC, H), jnp.float32),
)

_tc_fin = pl.pallas_call(
    _tc_fin_body,
    out_shape=jax.ShapeDtypeStruct((1, NCLS), jnp.float32),
)


def _r2(v):
    return v.reshape(1, -1)


def kernel(x, edge_index, gn0_w, gn0_b, gn0_ms, W1, b1, gn1_w, gn1_b, gn1_ms,
           W2, b2, gn2_w, gn2_b, gn2_ms, W3, b3, dense_W, dense_b, out_W, out_b):
    ei = edge_index.astype(jnp.int32)
    pad = E_PAD - E
    src_p = jnp.concatenate([ei[0], jnp.zeros((pad,), jnp.int32)])
    dst_p = jnp.concatenate([ei[1], jnp.full((pad,), N, jnp.int32)])
    src_blk = src_p.reshape(NW, NCH, CHUNK)
    dst_blk = dst_p.reshape(NW, NCH, CHUNK)
    zeros_acc = jnp.zeros((SLICE, HH), jnp.float32)
    ones_deg = jnp.ones((CHUNK, DEGW), jnp.float32)
    zeros_deg = jnp.zeros((NACC, DEGW), jnp.float32)

    degp = _deg_kernel(dst_blk, ones_deg, zeros_deg)

    h1 = _tc_norm_mm(x, _r2(gn0_w), _r2(gn0_b), _r2(gn0_ms), W1)
    g1, dinv = _tc_scale(h1, degp)
    sp1 = _agg_kernel(g1, src_blk, dst_blk, zeros_acc)
    g2 = _tc_mid(sp1, g1, dinv, _r2(b1), _r2(gn1_w), _r2(gn1_b),
                 _r2(gn1_ms), W2)
    sp2 = _agg_kernel(g2, src_blk, dst_blk, zeros_acc)
    g3 = _tc_mid(sp2, g2, dinv, _r2(b2), _r2(gn2_w), _r2(gn2_b),
                 _r2(gn2_ms), W3)
    sp3 = _agg_kernel(g3, src_blk, dst_blk, zeros_acc)
    return _tc_fin(sp3, g3, dinv, _r2(b3), dense_W, _r2(dense_b), out_W,
                   _r2(out_b))


# speedup vs baseline: 1.1687x; 1.0580x over previous
import functools

import jax
import jax.numpy as jnp
from jax import lax
from jax.experimental import pallas as pl
from jax.experimental.pallas import tpu as pltpu
from jax.experimental.pallas import tpu_sc as plsc

N = 10000
E = 320000
D = 128
H = 64
NCLS = 10

NC = 2
NS = 16
NW = NC * NS

CHUNK = 125
NCH = 80
EPT = NCH * CHUNK
NACC = 10112
SLICE = NACC // NS

_mesh = plsc.VectorSubcoreMesh(core_axis_name="c", subcore_axis_name="s")
_sc_params = pltpu.CompilerParams(use_tc_tiling_on_sc=False)


DEGW = 16


@functools.partial(
    pl.kernel,
    out_type=jax.ShapeDtypeStruct((NC, NACC, DEGW), jnp.float32),
    mesh=_mesh,
    scratch_types=[
        pltpu.VMEM((NCH, CHUNK), jnp.int32),
        pltpu.VMEM((CHUNK, DEGW), jnp.float32),
        pltpu.VMEM_SHARED((NACC, DEGW), jnp.float32),
        pltpu.SemaphoreType.DMA,
    ],
    compiler_params=_sc_params,
)
def _deg_kernel(dst_hbm, ones_hbm, zeros_hbm, out_hbm, dst_v, ones_v, acc, sem):
    cid = lax.axis_index("c")
    sid = lax.axis_index("s")
    wid = sid * NC + cid
    pltpu.sync_copy(dst_hbm.at[wid], dst_v)
    pltpu.sync_copy(ones_hbm, ones_v)
    pltpu.sync_copy(zeros_hbm.at[pl.ds(sid * SLICE, SLICE)],
                    acc.at[pl.ds(sid * SLICE, SLICE)])
    plsc.subcore_barrier()

    def body(j, carry):
        pltpu.async_copy(ones_v, acc.at[dst_v.at[j]], sem, add=True)
        return carry

    lax.fori_loop(0, NCH, body, 0)

    def drain(j, carry):
        pltpu.make_async_copy(ones_v, acc.at[dst_v.at[j]], sem).wait()
        return carry

    lax.fori_loop(0, NCH, drain, 0)
    plsc.subcore_barrier()
    pltpu.sync_copy(acc.at[pl.ds(sid * SLICE, SLICE)],
                    out_hbm.at[cid, pl.ds(sid * SLICE, SLICE)])


RING = 8
LAG = 4
HH = H // 2


@functools.partial(
    pl.kernel,
    out_type=jax.ShapeDtypeStruct((NC, NACC, H), jnp.float32),
    mesh=_mesh,
    scratch_types=[
        pltpu.VMEM((NCH, CHUNK), jnp.int32),
        pltpu.VMEM((NCH, CHUNK), jnp.int32),
        pltpu.VMEM((RING, CHUNK, HH), jnp.float32),
        pltpu.VMEM_SHARED((NACC, HH), jnp.float32),
        pltpu.VMEM_SHARED((NACC, HH), jnp.float32),
    ] + [pltpu.SemaphoreType.DMA] * (2 * RING),
    compiler_params=_sc_params,
)
def _agg_kernel(g_in, src_hbm, dst_hbm, zeros_hbm, out_hbm,
                src_v, dst_v, rows, acc, g_sh, *sems):
    gsem = sems[:RING]
    ssem = sems[RING:]
    cid = lax.axis_index("c")
    sid = lax.axis_index("s")
    wid = sid * NC + cid
    pltpu.sync_copy(src_hbm.at[wid], src_v)
    pltpu.sync_copy(dst_hbm.at[wid], dst_v)

    for p in range(2):
        pltpu.sync_copy(g_in.at[pl.ds(sid * SLICE, SLICE), pl.ds(p * HH, HH)],
                        g_sh.at[pl.ds(sid * SLICE, SLICE)])
        pltpu.sync_copy(zeros_hbm, acc.at[pl.ds(sid * SLICE, SLICE)])
        plsc.subcore_barrier()

        def g_issue(j, b):
            pltpu.async_copy(g_sh.at[src_v.at[j]], rows.at[b], gsem[b])

        def g_wait(j, b):
            pltpu.make_async_copy(g_sh.at[src_v.at[j]], rows.at[b],
                                  gsem[b]).wait()

        def s_issue(j, b):
            pltpu.async_copy(rows.at[b], acc.at[dst_v.at[j]], ssem[b],
                             add=True)

        def s_wait(j, b):
            pltpu.make_async_copy(rows.at[b], acc.at[dst_v.at[j]],
                                  ssem[b]).wait()

        for b in range(RING):
            g_issue(b, b)
        for j in range(LAG):
            g_wait(j, j)
            s_issue(j, j)

        def body(o, carry):
            for b in range(RING):
                i = o * RING + b
                s_wait(i - RING, b)
                g_issue(i, b)
                bp = (b - LAG) % RING
                g_wait(i - LAG, bp)
                s_issue(i - LAG, bp)
            return carry

        lax.fori_loop(1, NCH // RING, body, 0)

        for j in range(NCH - LAG, NCH):
            b = j % RING
            g_wait(j, b)
            s_issue(j, b)
        for j in range(NCH - RING, NCH):
            s_wait(j, j % RING)

        plsc.subcore_barrier()
        pltpu.sync_copy(acc.at[pl.ds(sid * SLICE, SLICE)],
                        out_hbm.at[cid, pl.ds(sid * SLICE, SLICE),
                                   pl.ds(p * HH, HH)])


_EPS = 1e-5


def _tc_norm_mm_body(x_ref, gnw, gnb, gnms, w1_ref, h1_ref):
    x = x_ref[...]
    mean = jnp.mean(x, axis=0, keepdims=True)
    xc = x - mean * gnms[...]
    var = jnp.mean(xc * xc, axis=0, keepdims=True)
    hn = xc * jax.lax.rsqrt(var + _EPS) * gnw[...] + gnb[...]
    h1_ref[...] = jnp.dot(hn, w1_ref[...], preferred_element_type=jnp.float32)


def _tc_scale_body(h1_ref, degp_ref, g_ref, dinv_ref):
    deg = degp_ref[0, :N, 0:1] + degp_ref[1, :N, 0:1] + 1.0
    dinv = jax.lax.rsqrt(deg)
    dinv_ref[...] = dinv
    g_ref[:N, :] = h1_ref[...] * dinv
    g_ref[N:, :] = jnp.zeros((NACC - N, H), jnp.float32)


def _tc_mid_body(sp_ref, g_ref, dinv_ref, b_ref,
                 gnw, gnb, gnms, w_ref, g_out):
    s = sp_ref[0, :N, :] + sp_ref[1, :N, :] + g_ref[:N, :]
    dinv = dinv_ref[...]
    h = jnp.maximum(dinv * s + b_ref[...], 0.0)
    mean = jnp.mean(h, axis=0, keepdims=True)
    hc = h - mean * gnms[...]
    var = jnp.mean(hc * hc, axis=0, keepdims=True)
    hn = hc * jax.lax.rsqrt(var + _EPS) * gnw[...] + gnb[...]
    h2 = jnp.dot(hn, w_ref[...], preferred_element_type=jnp.float32)
    g_out[:N, :] = h2 * dinv
    g_out[N:, :] = jnp.zeros((NACC - N, H), jnp.float32)


def _tc_fin_body(sp_ref, g_ref, dinv_ref, b_ref,
                 dw_ref, db_ref, ow_ref, ob_ref, out_ref):
    s = sp_ref[0, :N, :] + sp_ref[1, :N, :] + g_ref[:N, :]
    h = jnp.maximum(dinv_ref[...] * s + b_ref[...], 0.0)
    gm = jnp.mean(h, axis=0, keepdims=True)
    dh = jnp.maximum(
        jnp.dot(gm, dw_ref[...], preferred_element_type=jnp.float32) + db_ref[...],
        0.0)
    logits = jnp.dot(dh, ow_ref[...], preferred_element_type=jnp.float32) + ob_ref[...]
    e = jnp.exp(logits - jnp.max(logits, axis=1, keepdims=True))
    out_ref[...] = e / jnp.sum(e, axis=1, keepdims=True)


_tc_norm_mm = pl.pallas_call(
    _tc_norm_mm_body,
    out_shape=jax.ShapeDtypeStruct((N, H), jnp.float32),
)

_tc_scale = pl.pallas_call(
    _tc_scale_body,
    out_shape=(jax.ShapeDtypeStruct((NACC, H), jnp.float32),
               jax.ShapeDtypeStruct((N, 1), jnp.float32)),
)

_tc_mid = pl.pallas_call(
    _tc_mid_body,
    out_shape=jax.ShapeDtypeStruct((NACC, H), jnp.float32),
)

_tc_fin = pl.pallas_call(
    _tc_fin_body,
    out_shape=jax.ShapeDtypeStruct((1, NCLS), jnp.float32),
)


def _r2(v):
    return v.reshape(1, -1)


def kernel(x, edge_index, gn0_w, gn0_b, gn0_ms, W1, b1, gn1_w, gn1_b, gn1_ms,
           W2, b2, gn2_w, gn2_b, gn2_ms, W3, b3, dense_W, dense_b, out_W, out_b):
    ei = edge_index.astype(jnp.int32)
    src_blk = ei[0].reshape(NW, NCH, CHUNK)
    dst_blk = ei[1].reshape(NW, NCH, CHUNK)
    zeros_acc = jnp.zeros((SLICE, HH), jnp.float32)
    ones_deg = jnp.ones((CHUNK, DEGW), jnp.float32)
    zeros_deg = jnp.zeros((NACC, DEGW), jnp.float32)

    degp = _deg_kernel(dst_blk, ones_deg, zeros_deg)

    h1 = _tc_norm_mm(x, _r2(gn0_w), _r2(gn0_b), _r2(gn0_ms), W1)
    g1, dinv = _tc_scale(h1, degp)
    sp1 = _agg_kernel(g1, src_blk, dst_blk, zeros_acc)
    g2 = _tc_mid(sp1, g1, dinv, _r2(b1), _r2(gn1_w), _r2(gn1_b),
                 _r2(gn1_ms), W2)
    sp2 = _agg_kernel(g2, src_blk, dst_blk, zeros_acc)
    g3 = _tc_mid(sp2, g2, dinv, _r2(b2), _r2(gn2_w), _r2(gn2_b),
                 _r2(gn2_ms), W3)
    sp3 = _agg_kernel(g3, src_blk, dst_blk, zeros_acc)
    return _tc_fin(sp3, g3, dinv, _r2(b3), dense_W, _r2(dense_b), out_W,
                   _r2(out_b))
